# baseline (speedup 1.0000x reference)
import functools

import jax
import jax.numpy as jnp
from jax import lax
from jax.experimental import pallas as pl
from jax.experimental.pallas import tpu as pltpu
from jax.experimental.pallas import tpu_sc as plsc

POOL = 7
H = 56
W = 56
C = 768
LANES = 16
CB = C // LANES
MAXSPAN = 35
KMID = 5
KLAST = 10
NROI = 32
OUTSZ = POOL * POOL * C
NEG = -3.0e38
WSPLIT = 3
MAXL0 = 3 * KMID
MAXL1 = MAXSPAN - 3 * 2
NSC = 32


def _mesh():
    return plsc.VectorSubcoreMesh(core_axis_name="c", subcore_axis_name="s")


@functools.partial(
    pl.kernel,
    out_type=jax.ShapeDtypeStruct((NROI * OUTSZ,), jnp.float32),
    mesh=_mesh(),
    scratch_types=[
        pltpu.VMEM((LANES,), jnp.int32),
        pltpu.VMEM((MAXL1 * LANES,), jnp.int32),
        pltpu.VMEM((MAXSPAN * C,), jnp.float32),
        pltpu.VMEM((MAXSPAN * C,), jnp.float32),
        pltpu.VMEM((POOL * WSPLIT * C,), jnp.float32),
        pltpu.VMEM((POOL * (POOL - WSPLIT) * C,), jnp.float32),
        pltpu.SemaphoreType.DMA,
        pltpu.SemaphoreType.DMA,
    ],
)
def _roi_sc(feat_hbm, params_hbm, xtab_hbm, out_hbm,
            pbuf, xtab, line0, line1, oacc0, oacc1, sem0, sem1):
    cid = lax.axis_index("c")
    sid = lax.axis_index("s")
    wid = cid * 16 + sid

    line_bufs = (line0, line1)
    sems = (sem0, sem1)
    neg_vec = jnp.full((LANES,), NEG, dtype=jnp.float32)

    for slot, wcnt, maxl, oacc in ((0, WSPLIT, MAXL0, oacc0),
                                   (1, POOL - WSPLIT, MAXL1, oacc1)):
        item = slot * NSC + wid
        pltpu.sync_copy(
            params_hbm.at[pl.ds(pl.multiple_of(item * LANES, LANES), LANES)], pbuf)
        pltpu.sync_copy(
            xtab_hbm.at[pl.ds(pl.multiple_of(item * (MAXL1 * LANES), LANES),
                              maxl * LANES)],
            xtab.at[pl.ds(0, maxl * LANES)])

        p = pbuf[pl.ds(0, LANES)]
        x0, n, base, outb = p[0], p[1], p[2], p[3]
        ryb = [p[4 + i] for i in range(8)]

        rofs = []
        for h in range(POOL):
            kmax = KLAST if h == POOL - 1 else KMID
            rofs.append([jnp.minimum(ryb[h] + k, ryb[h + 1] - 1) * C
                         for k in range(kmax)])

        def init_i(i, _, oacc=oacc):
            for u in range(8):
                oacc[pl.ds((i * 8 + u) * LANES, LANES)] = neg_vec
            return 0

        lax.fori_loop(0, POOL * wcnt * CB // 8, init_i, 0)

        def _start(j, par, base=base, x0=x0):
            off = pl.multiple_of(base + (x0 + j) * (W * C), C)
            pltpu.make_async_copy(
                feat_hbm.at[pl.ds(off, MAXSPAN * C)], line_bufs[par], sems[par]
            ).start()

        def _wait(par):
            pltpu.make_async_copy(
                feat_hbm.at[pl.ds(0, MAXSPAN * C)], line_bufs[par], sems[par]
            ).wait()

        _start(0, 0)
        _start(1, 1)

        def _line(j, par, n=n, wcnt=wcnt, rofs=rofs, oacc=oacc):
            _wait(par)
            line = line_bufs[par]
            ow = xtab[pl.ds(pl.multiple_of(j * LANES, LANES), LANES)][0]
            for h in range(POOL):
                obase = h * (wcnt * C) + ow
                offs = rofs[h]

                def cbody(cb, _, obase=obase, offs=offs):
                    c0 = pl.multiple_of(cb * LANES, LANES)
                    acc = oacc[pl.ds(obase + c0, LANES)]
                    for o in offs:
                        acc = jnp.maximum(acc, line[pl.ds(o + c0, LANES)])
                    oacc[pl.ds(obase + c0, LANES)] = acc
                    return 0

                lax.fori_loop(0, CB, cbody, 0)

            @pl.when(j + 2 < n)
            def _():
                _start(j + 2, par)

        def pair(j2, _):
            j0 = j2 * 2
            _line(j0, 0)

            @pl.when(j0 + 1 < n)
            def _():
                _line(j0 + 1, 1)

            return 0

        lax.fori_loop(0, (n + 1) // 2, pair, 0)

        for h in range(POOL):
            pltpu.sync_copy(
                oacc.at[pl.ds(h * (wcnt * C), wcnt * C)],
                out_hbm.at[pl.ds(pl.multiple_of(outb + h * (POOL * C), C),
                                 wcnt * C)])


def kernel(features, rois):
    B, N = rois.shape[0], rois.shape[1]
    r = rois.astype(jnp.int32).reshape(NROI, 4)
    minx, miny, maxx, maxy = r[:, 0], r[:, 1], r[:, 2], r[:, 3]
    dx = (maxx - minx) // POOL
    dy = (maxy - miny) // POOL
    nx = maxx - minx
    span = maxy - miny
    k = jnp.arange(POOL, dtype=jnp.int32)
    yb = jnp.concatenate([miny[:, None] + k[None, :] * dy[:, None], maxy[:, None]], axis=1)
    cstart = jnp.minimum(miny, W - MAXSPAN)
    ryb = yb - cstart[:, None]
    b_of = jnp.arange(NROI, dtype=jnp.int32) // N
    base = (b_of * (H * W) + cstart) * C
    roi_out = jnp.arange(NROI, dtype=jnp.int32) * OUTSZ

    n0 = WSPLIT * dx
    n1 = nx - n0
    x0_f, x0_b = minx, minx + n0
    outb_f, outb_b = roi_out, roi_out + WSPLIT * C

    j = jnp.arange(MAXL1, dtype=jnp.int32)
    wl_f = jnp.minimum(j[None, :] // dx[:, None], WSPLIT - 1)
    wl_b = jnp.minimum((n0[:, None] + j[None, :]) // dx[:, None], POOL - 1) - WSPLIT

    def pack(x0, n, outb, wl):
        prm = jnp.zeros((NROI, LANES), jnp.int32)
        prm = (prm.at[:, 0].set(x0).at[:, 1].set(n).at[:, 2].set(base)
               .at[:, 3].set(outb).at[:, 4:12].set(ryb))
        xt = jnp.zeros((NROI, MAXL1, LANES), jnp.int32)
        xt = xt.at[:, :, 0].set(wl * C)
        return prm, xt

    prm_f, xt_f = pack(x0_f, n0, outb_f, wl_f)
    prm_b, xt_b = pack(x0_b, n1, outb_b, wl_b)

    o_f = jnp.argsort(-(n0 * span))
    o_b = jnp.argsort(n1 * span)
    params = jnp.concatenate([prm_f[o_f], prm_b[o_b]], axis=0)
    xtab = jnp.concatenate([xt_f[o_f], xt_b[o_b]], axis=0)

    feat_flat = features.reshape(B * H * W * C)
    out = _roi_sc(feat_flat, params.reshape(-1), xtab.reshape(-1))
    return out.reshape(B, N, POOL, POOL, C)


# speedup vs baseline: 1.3497x; 1.3497x over previous
import functools

import jax
import jax.numpy as jnp
from jax import lax
from jax.experimental import pallas as pl
from jax.experimental.pallas import tpu as pltpu
from jax.experimental.pallas import tpu_sc as plsc

POOL = 7
H = 56
W = 56
C = 768
LANES = 16
CB = C // LANES
MAXSPAN = 35
KMID = 5
KLAST = 10
NROI = 32
OUTSZ = POOL * POOL * C
NEG = -3.0e38
WSPLIT = 3
MAXL0 = 3 * KMID
MAXL1 = MAXSPAN - 3 * 2
NSC = 32


def _mesh():
    return plsc.VectorSubcoreMesh(core_axis_name="c", subcore_axis_name="s")


@functools.partial(
    pl.kernel,
    out_type=jax.ShapeDtypeStruct((NROI * OUTSZ,), jnp.float32),
    mesh=_mesh(),
    scratch_types=[
        pltpu.VMEM((LANES,), jnp.int32),
        pltpu.VMEM((MAXL1 * LANES,), jnp.int32),
        pltpu.VMEM((MAXSPAN * C,), jnp.float32),
        pltpu.VMEM((MAXSPAN * C,), jnp.float32),
        pltpu.VMEM((POOL * WSPLIT * C,), jnp.float32),
        pltpu.VMEM((POOL * (POOL - WSPLIT) * C,), jnp.float32),
        pltpu.SemaphoreType.DMA,
        pltpu.SemaphoreType.DMA,
    ],
)
def _roi_sc(feat_hbm, params_hbm, xtab_hbm, out_hbm,
            pbuf, xtab, line0, line1, oacc0, oacc1, sem0, sem1):
    cid = lax.axis_index("c")
    sid = lax.axis_index("s")
    wid = cid * 16 + sid

    line_bufs = (line0, line1)
    sems = (sem0, sem1)
    neg_vec = jnp.full((LANES,), NEG, dtype=jnp.float32)

    for slot, wcnt, maxl, oacc in ((0, WSPLIT, MAXL0, oacc0),
                                   (1, POOL - WSPLIT, MAXL1, oacc1)):
        item = slot * NSC + wid
        pltpu.sync_copy(
            params_hbm.at[pl.ds(pl.multiple_of(item * LANES, LANES), LANES)], pbuf)
        pltpu.sync_copy(
            xtab_hbm.at[pl.ds(pl.multiple_of(item * (MAXL1 * LANES), LANES),
                              maxl * LANES)],
            xtab.at[pl.ds(0, maxl * LANES)])

        p = pbuf[pl.ds(0, LANES)]
        x0, n, base, outb = p[0], p[1], p[2], p[3]
        ryb = [p[4 + i] for i in range(8)]

        rofs = []
        for h in range(POOL):
            kmax = KLAST if h == POOL - 1 else KMID
            rofs.append([jnp.minimum(ryb[h] + k, ryb[h + 1] - 1) * C
                         for k in range(kmax)])

        def init_i(i, _, oacc=oacc):
            for u in range(8):
                oacc[pl.ds((i * 8 + u) * LANES, LANES)] = neg_vec
            return 0

        lax.fori_loop(0, POOL * wcnt * CB // 8, init_i, 0)

        def _start(j, par, base=base, x0=x0):
            off = pl.multiple_of(base + (x0 + j) * (W * C), C)
            pltpu.make_async_copy(
                feat_hbm.at[pl.ds(off, MAXSPAN * C)], line_bufs[par], sems[par]
            ).start()

        def _wait(par):
            pltpu.make_async_copy(
                feat_hbm.at[pl.ds(0, MAXSPAN * C)], line_bufs[par], sems[par]
            ).wait()

        _start(0, 0)
        _start(1, 1)

        def _line(j, par, n=n, wcnt=wcnt, rofs=rofs, oacc=oacc):
            _wait(par)
            line = line_bufs[par]
            ow = xtab[pl.ds(pl.multiple_of(j * LANES, LANES), LANES)][0]

            @plsc.parallel_loop(0, CB, step=1, unroll=2)
            def cbody(cb, wcnt=wcnt, rofs=rofs, oacc=oacc, line=line, ow=ow):
                c0 = pl.multiple_of(cb * LANES, LANES)
                for h in range(POOL):
                    obase = h * (wcnt * C) + ow
                    vals = [line[pl.ds(o + c0, LANES)] for o in rofs[h]]
                    vals.append(oacc[pl.ds(obase + c0, LANES)])
                    while len(vals) > 1:
                        vals = [jnp.maximum(a, b) for a, b in zip(vals[::2], vals[1::2])] \
                            + ([vals[-1]] if len(vals) % 2 else [])
                    oacc[pl.ds(obase + c0, LANES)] = vals[0]

            @pl.when(j + 2 < n)
            def _():
                _start(j + 2, par)

        def pair(j2, _):
            j0 = j2 * 2
            _line(j0, 0)

            @pl.when(j0 + 1 < n)
            def _():
                _line(j0 + 1, 1)

            return 0

        lax.fori_loop(0, (n + 1) // 2, pair, 0)

        for h in range(POOL):
            pltpu.sync_copy(
                oacc.at[pl.ds(h * (wcnt * C), wcnt * C)],
                out_hbm.at[pl.ds(pl.multiple_of(outb + h * (POOL * C), C),
                                 wcnt * C)])


def kernel(features, rois):
    B, N = rois.shape[0], rois.shape[1]
    r = rois.astype(jnp.int32).reshape(NROI, 4)
    minx, miny, maxx, maxy = r[:, 0], r[:, 1], r[:, 2], r[:, 3]
    dx = (maxx - minx) // POOL
    dy = (maxy - miny) // POOL
    nx = maxx - minx
    span = maxy - miny
    k = jnp.arange(POOL, dtype=jnp.int32)
    yb = jnp.concatenate([miny[:, None] + k[None, :] * dy[:, None], maxy[:, None]], axis=1)
    cstart = jnp.minimum(miny, W - MAXSPAN)
    ryb = yb - cstart[:, None]
    b_of = jnp.arange(NROI, dtype=jnp.int32) // N
    base = (b_of * (H * W) + cstart) * C
    roi_out = jnp.arange(NROI, dtype=jnp.int32) * OUTSZ

    n0 = WSPLIT * dx
    n1 = nx - n0
    x0_f, x0_b = minx, minx + n0
    outb_f, outb_b = roi_out, roi_out + WSPLIT * C

    j = jnp.arange(MAXL1, dtype=jnp.int32)
    wl_f = jnp.minimum(j[None, :] // dx[:, None], WSPLIT - 1)
    wl_b = jnp.minimum((n0[:, None] + j[None, :]) // dx[:, None], POOL - 1) - WSPLIT

    def pack(x0, n, outb, wl):
        prm = jnp.zeros((NROI, LANES), jnp.int32)
        prm = (prm.at[:, 0].set(x0).at[:, 1].set(n).at[:, 2].set(base)
               .at[:, 3].set(outb).at[:, 4:12].set(ryb))
        xt = jnp.zeros((NROI, MAXL1, LANES), jnp.int32)
        xt = xt.at[:, :, 0].set(wl * C)
        return prm, xt

    prm_f, xt_f = pack(x0_f, n0, outb_f, wl_f)
    prm_b, xt_b = pack(x0_b, n1, outb_b, wl_b)

    o_f = jnp.argsort(-n0)
    o_b = jnp.argsort(n1)
    params = jnp.concatenate([prm_f[o_f], prm_b[o_b]], axis=0)
    xtab = jnp.concatenate([xt_f[o_f], xt_b[o_b]], axis=0)

    feat_flat = features.reshape(B * H * W * C)
    out = _roi_sc(feat_flat, params.reshape(-1), xtab.reshape(-1))
    return out.reshape(B, N, POOL, POOL, C)


# speedup vs baseline: 1.3999x; 1.0371x over previous
import functools

import jax
import jax.numpy as jnp
from jax import lax
from jax.experimental import pallas as pl
from jax.experimental.pallas import tpu as pltpu
from jax.experimental.pallas import tpu_sc as plsc

POOL = 7
H = 56
W = 56
C = 768
LANES = 16
CB = C // LANES
MAXSPAN = 35
NROI = 32
OUTSZ = POOL * POOL * C
NEG = -3.0e38
WSPLIT = 3
MAXL1 = 29
NSC = 32

_VARIANTS = ((2, 8, 20, False), (3, 9, 27, False), (5, 10, 35, True))


def _mesh():
    return plsc.VectorSubcoreMesh(core_axis_name="c", subcore_axis_name="s")


@functools.partial(
    pl.kernel,
    out_type=jax.ShapeDtypeStruct((NROI * OUTSZ,), jnp.float32),
    mesh=_mesh(),
    scratch_types=[
        pltpu.VMEM((LANES,), jnp.int32),
        pltpu.VMEM((MAXL1 * LANES,), jnp.int32),
        pltpu.VMEM((MAXSPAN * C,), jnp.float32),
        pltpu.VMEM((MAXSPAN * C,), jnp.float32),
        pltpu.VMEM((POOL * WSPLIT * C,), jnp.float32),
        pltpu.VMEM((POOL * (POOL - WSPLIT) * C,), jnp.float32),
        pltpu.SemaphoreType.DMA,
        pltpu.SemaphoreType.DMA,
    ],
)
def _roi_sc(feat_hbm, params_hbm, xtab_hbm, out_hbm,
            pbuf, xtab, line0, line1, oacc0, oacc1, sem0, sem1):
    cid = lax.axis_index("c")
    sid = lax.axis_index("s")
    wid = cid * 16 + sid

    line_bufs = (line0, line1)
    sems = (sem0, sem1)
    neg_vec = jnp.full((LANES,), NEG, dtype=jnp.float32)

    for slot, wcnt, oacc in ((0, WSPLIT, oacc0), (1, POOL - WSPLIT, oacc1)):
        item = slot * NSC + wid
        pltpu.sync_copy(
            params_hbm.at[pl.ds(pl.multiple_of(item * LANES, LANES), LANES)], pbuf)
        pltpu.sync_copy(
            xtab_hbm.at[pl.ds(pl.multiple_of(item * (MAXL1 * LANES), LANES),
                              MAXL1 * LANES)], xtab)

        p = pbuf[pl.ds(0, LANES)]
        x0, n, base, outb = p[0], p[1], p[2], p[3]
        ryb = [p[4 + i] for i in range(8)]
        dyv = p[12]

        def init_i(i, _, oacc=oacc):
            for u in range(8):
                oacc[pl.ds((i * 8 + u) * LANES, LANES)] = neg_vec
            return 0

        lax.fori_loop(0, POOL * wcnt * CB // 8, init_i, 0)

        for vi, (km, kl, spanv, clamp_mid) in enumerate(_VARIANTS):
            cond = dyv >= 4 if vi == 2 else dyv == km

            @pl.when(cond)
            def _(km=km, kl=kl, spanv=spanv, clamp_mid=clamp_mid,
                  wcnt=wcnt, oacc=oacc, x0=x0, n=n, base=base):
                rofs = []
                for h in range(POOL - 1):
                    if clamp_mid:
                        rofs.append([jnp.minimum(ryb[h] + k, ryb[h + 1] - 1) * C
                                     for k in range(km)])
                    else:
                        rofs.append([(ryb[h] + k) * C for k in range(km)])
                rofs.append([jnp.minimum(ryb[POOL - 1] + k, ryb[POOL] - 1) * C
                             for k in range(kl)])

                def _start(j, par):
                    off = pl.multiple_of(base + (x0 + j) * (W * C), C)
                    pltpu.make_async_copy(
                        feat_hbm.at[pl.ds(off, spanv * C)],
                        line_bufs[par].at[pl.ds(0, spanv * C)], sems[par]
                    ).start()

                def _wait(par):
                    pltpu.make_async_copy(
                        feat_hbm.at[pl.ds(0, spanv * C)],
                        line_bufs[par].at[pl.ds(0, spanv * C)], sems[par]
                    ).wait()

                _start(0, 0)
                _start(1, 1)

                def _line(j, par):
                    _wait(par)
                    line = line_bufs[par]
                    ow = xtab[pl.ds(pl.multiple_of(j * LANES, LANES), LANES)][0]

                    @plsc.parallel_loop(0, CB, step=1, unroll=2)
                    def cbody(cb):
                        c0 = pl.multiple_of(cb * LANES, LANES)
                        for h in range(POOL):
                            obase = h * (wcnt * C) + ow
                            vals = [line[pl.ds(o + c0, LANES)] for o in rofs[h]]
                            vals.append(oacc[pl.ds(obase + c0, LANES)])
                            while len(vals) > 1:
                                vals = ([jnp.maximum(a, b)
                                         for a, b in zip(vals[::2], vals[1::2])]
                                        + ([vals[-1]] if len(vals) % 2 else []))
                            oacc[pl.ds(obase + c0, LANES)] = vals[0]

                    @pl.when(j + 2 < n)
                    def _():
                        _start(j + 2, par)

                def pair(j2, _):
                    j0 = j2 * 2
                    _line(j0, 0)

                    @pl.when(j0 + 1 < n)
                    def _():
                        _line(j0 + 1, 1)

                    return 0

                lax.fori_loop(0, (n + 1) // 2, pair, 0)

        for h in range(POOL):
            pltpu.sync_copy(
                oacc.at[pl.ds(h * (wcnt * C), wcnt * C)],
                out_hbm.at[pl.ds(pl.multiple_of(outb + h * (POOL * C), C),
                                 wcnt * C)])


def kernel(features, rois):
    B, N = rois.shape[0], rois.shape[1]
    r = rois.astype(jnp.int32).reshape(NROI, 4)
    minx, miny, maxx, maxy = r[:, 0], r[:, 1], r[:, 2], r[:, 3]
    dx = (maxx - minx) // POOL
    dy = (maxy - miny) // POOL
    nx = maxx - minx
    k = jnp.arange(POOL, dtype=jnp.int32)
    yb = jnp.concatenate([miny[:, None] + k[None, :] * dy[:, None], maxy[:, None]], axis=1)
    spanv = jnp.where(dy == 2, 20, jnp.where(dy == 3, 27, MAXSPAN))
    cstart = jnp.minimum(miny, W - spanv)
    ryb = yb - cstart[:, None]
    b_of = jnp.arange(NROI, dtype=jnp.int32) // N
    base = (b_of * (H * W) + cstart) * C
    roi_out = jnp.arange(NROI, dtype=jnp.int32) * OUTSZ

    n0 = WSPLIT * dx
    n1 = nx - n0
    x0_f, x0_b = minx, minx + n0
    outb_f, outb_b = roi_out, roi_out + WSPLIT * C

    j = jnp.arange(MAXL1, dtype=jnp.int32)
    wl_f = jnp.minimum(j[None, :] // dx[:, None], WSPLIT - 1)
    wl_b = jnp.minimum((n0[:, None] + j[None, :]) // dx[:, None], POOL - 1) - WSPLIT

    def pack(x0, n, outb, wl):
        prm = jnp.zeros((NROI, LANES), jnp.int32)
        prm = (prm.at[:, 0].set(x0).at[:, 1].set(n).at[:, 2].set(base)
               .at[:, 3].set(outb).at[:, 4:12].set(ryb).at[:, 12].set(dy))
        xt = jnp.zeros((NROI, MAXL1, LANES), jnp.int32)
        xt = xt.at[:, :, 0].set(wl * C)
        return prm, xt

    prm_f, xt_f = pack(x0_f, n0, outb_f, wl_f)
    prm_b, xt_b = pack(x0_b, n1, outb_b, wl_b)

    loads = jnp.where(dy == 2, 27, jnp.where(dy == 3, 34, 47))
    o_f = jnp.argsort(-(n0 * loads))
    o_b = jnp.argsort(n1 * loads)
    params = jnp.concatenate([prm_f[o_f], prm_b[o_b]], axis=0)
    xtab = jnp.concatenate([xt_f[o_f], xt_b[o_b]], axis=0)

    feat_flat = features.reshape(B * H * W * C)
    out = _roi_sc(feat_flat, params.reshape(-1), xtab.reshape(-1))
    return out.reshape(B, N, POOL, POOL, C)
